# baseline (speedup 1.0000x reference)
import functools

import jax
import jax.numpy as jnp
from jax import lax
from jax.experimental import pallas as pl
from jax.experimental.pallas import tpu as pltpu
from jax.experimental.pallas import tpu_sc as plsc

B, N, K, C = 8, 2048, 16, 256
CS = 64
HC = 32
NH = 512
NSL = C // CS
NTASK = B * NSL


def _mm_body(f_ref, w_ref, o_ref):
    o_ref[...] = lax.dot_general(
        f_ref[...].astype(jnp.bfloat16), w_ref[...].astype(jnp.bfloat16),
        (((1,), (1,)), ((), ())),
        preferred_element_type=jnp.float32).astype(jnp.bfloat16)


def _matmul(f2d, W):
    blk = 16384
    return pl.pallas_call(
        _mm_body,
        grid=(f2d.shape[0] // blk,),
        in_specs=[
            pl.BlockSpec((blk, C), lambda i: (i, 0)),
            pl.BlockSpec((C, C), lambda i: (0, 0)),
        ],
        out_specs=pl.BlockSpec((blk, C), lambda i: (i, 0)),
        out_shape=jax.ShapeDtypeStruct((f2d.shape[0], C), jnp.bfloat16),
    )(f2d, W)


def _split_f32(pb):
    w = plsc.bitcast(pb, jnp.int32)
    lo = plsc.bitcast(lax.shift_left(w, 16), jnp.float32)
    hi = plsc.bitcast(w & jnp.int32(-65536), jnp.float32)
    return lo, hi


@functools.partial(
    pl.kernel,
    out_type=[
        jax.ShapeDtypeStruct((B * N, C), jnp.bfloat16),
        jax.ShapeDtypeStruct((B, 2, C), jnp.float32),
    ],
    mesh=plsc.VectorSubcoreMesh(core_axis_name="c", subcore_axis_name="s"),
    compiler_params=pltpu.CompilerParams(use_tc_tiling_on_sc=False,
                                         needs_layout_passes=False),
    scratch_types=[
        pltpu.VMEM((N, CS), jnp.bfloat16),
        pltpu.VMEM((2, (NH + 1) * K), jnp.int32),
        pltpu.VMEM((2, NH, CS), jnp.bfloat16),
        pltpu.VMEM((2, CS), jnp.float32),
        pltpu.SemaphoreType.DMA,
        pltpu.SemaphoreType.DMA,
        pltpu.SemaphoreType.DMA,
        pltpu.SemaphoreType.DMA,
        pltpu.SemaphoreType.DMA,
    ],
)
def _sc_gather_max(x_hbm, idx_hbm, pooled_hbm, part_hbm,
                   xs_v, idx_v, pooled_v, stats_v,
                   sem_x, sem_i0, sem_i1, sem_o0, sem_o1):
    t = lax.axis_index("s") * 2 + lax.axis_index("c")
    b = t // NSL
    c0 = (t % NSL) * CS
    sem_i = (sem_i0, sem_i1)
    sem_o = (sem_o0, sem_o1)
    nch = N // NH

    def idx_fetch(h):
        return pltpu.async_copy(
            idx_hbm.at[pl.ds((b * N + h * NH) * K, NH * K)],
            idx_v.at[h % 2, pl.ds(0, NH * K)], sem_i[h % 2])

    cp_x = pltpu.async_copy(x_hbm.at[pl.ds(b * N, N), pl.ds(c0, CS)],
                            xs_v, sem_x)
    cp_i = idx_fetch(0)
    cp_x.wait()
    z = jnp.zeros((16,), jnp.float32)
    stats = (z,) * 8
    wb = [None, None]
    for h in range(nch):
        cur = h % 2
        cp_i.wait()
        if h + 1 < nch:
            cp_i = idx_fetch(h + 1)
        if wb[cur] is not None:
            wb[cur].wait()
        iv0 = idx_v[cur, pl.ds(0, K)]
        init = tuple(iv0[j] for j in range(K))

        def body(n, carry, h=h, cur=cur):
            a, st = carry
            ivn = idx_v[cur, pl.ds((n + 1) * K, K)]
            nxt = tuple(ivn[j] for j in range(K))
            m0 = xs_v[a[0], pl.ds(0, HC)]
            m1 = xs_v[a[0], pl.ds(HC, HC)]
            for j in range(1, K):
                m0 = jnp.maximum(m0, xs_v[a[j], pl.ds(0, HC)])
                m1 = jnp.maximum(m1, xs_v[a[j], pl.ds(HC, HC)])
            sn = h * NH + n
            p0 = m0 - xs_v[sn, pl.ds(0, HC)]
            p1 = m1 - xs_v[sn, pl.ds(HC, HC)]
            pooled_v[cur, n, pl.ds(0, HC)] = p0
            pooled_v[cur, n, pl.ds(HC, HC)] = p1
            e0, o0 = _split_f32(p0)
            e1, o1 = _split_f32(p1)
            se0, so0, se1, so1, qe0, qo0, qe1, qo1 = st
            st = (se0 + e0, so0 + o0, se1 + e1, so1 + o1,
                  qe0 + e0 * e0, qo0 + o0 * o0, qe1 + e1 * e1, qo1 + o1 * o1)
            return (nxt, st)

        _, stats = lax.fori_loop(0, NH, body, (init, stats))
        wb[cur] = pltpu.async_copy(
            pooled_v.at[cur],
            pooled_hbm.at[pl.ds(b * N + h * NH, NH), pl.ds(c0, CS)],
            sem_o[cur])
    for w in wb:
        w.wait()
    ii = jnp.arange(16, dtype=jnp.int32)
    ev, od = ii * 2, ii * 2 + 1
    r0, r1 = ii * 0, ii * 0 + 1
    se0, so0, se1, so1, qe0, qo0, qe1, qo1 = stats
    plsc.store_scatter(stats_v, [r0, ev], se0)
    plsc.store_scatter(stats_v, [r0, od], so0)
    plsc.store_scatter(stats_v, [r0, ev + HC], se1)
    plsc.store_scatter(stats_v, [r0, od + HC], so1)
    plsc.store_scatter(stats_v, [r1, ev], qe0)
    plsc.store_scatter(stats_v, [r1, od], qo0)
    plsc.store_scatter(stats_v, [r1, ev + HC], qe1)
    plsc.store_scatter(stats_v, [r1, od + HC], qo1)
    pltpu.sync_copy(stats_v, part_hbm.at[b, :, pl.ds(c0, CS)])


def _bn_body(pooled_ref, part_ref, w_ref, b_ref, o_ref):
    cnt = float(B * N)
    tot = jnp.sum(part_ref[...], axis=0)
    mean = tot[0:1, :] / cnt
    var = tot[1:2, :] / cnt - mean * mean
    inv = lax.rsqrt(var + 1e-5)
    o_ref[...] = (pooled_ref[...].astype(jnp.float32) - mean) \
        * (inv * w_ref[...]) + b_ref[...]


def _batchnorm(pooled2d, partials, bnw2d, bnb2d):
    blk = 8192
    nb = pooled2d.shape[0] // blk
    return pl.pallas_call(
        _bn_body,
        grid=(nb,),
        in_specs=[
            pl.BlockSpec((blk, C), lambda i: (i, 0)),
            pl.BlockSpec((B, 2, C), lambda i: (0, 0, 0)),
            pl.BlockSpec((1, C), lambda i: (0, 0)),
            pl.BlockSpec((1, C), lambda i: (0, 0)),
        ],
        out_specs=pl.BlockSpec((blk, C), lambda i: (i, 0)),
        out_shape=jax.ShapeDtypeStruct((pooled2d.shape[0], C), jnp.float32),
    )(pooled2d, partials, bnw2d, bnb2d)


def kernel(f, group_idx, W, bn_weight, bn_bias):
    x = _matmul(f.reshape(B * N, C), W)
    pooled, partials = _sc_gather_max(x, group_idx.reshape(B * N * K))
    out = _batchnorm(pooled, partials,
                     bn_weight.reshape(1, C), bn_bias.reshape(1, C))
    return out.reshape(B, N, C)


# speedup vs baseline: 1.0054x; 1.0054x over previous
import functools

import jax
import jax.numpy as jnp
from jax import lax
from jax.experimental import pallas as pl
from jax.experimental.pallas import tpu as pltpu
from jax.experimental.pallas import tpu_sc as plsc

B, N, K, C = 8, 2048, 16, 256
CS = 64
HC = 32
NH = 512
NSL = C // CS
NTASK = B * NSL


def _mm_body(f_ref, w_ref, o_ref):
    o_ref[...] = lax.dot_general(
        f_ref[...].astype(jnp.bfloat16), w_ref[...].astype(jnp.bfloat16),
        (((1,), (1,)), ((), ())),
        preferred_element_type=jnp.float32).astype(jnp.bfloat16)


def _matmul(f2d, W):
    blk = 8192
    return pl.pallas_call(
        _mm_body,
        grid=(f2d.shape[0] // blk,),
        in_specs=[
            pl.BlockSpec((blk, C), lambda i: (i, 0)),
            pl.BlockSpec((C, C), lambda i: (0, 0)),
        ],
        out_specs=pl.BlockSpec((blk, C), lambda i: (i, 0)),
        out_shape=jax.ShapeDtypeStruct((f2d.shape[0], C), jnp.bfloat16),
    )(f2d, W)


def _split_f32(pb):
    w = plsc.bitcast(pb, jnp.int32)
    lo = plsc.bitcast(lax.shift_left(w, 16), jnp.float32)
    hi = plsc.bitcast(w & jnp.int32(-65536), jnp.float32)
    return lo, hi


@functools.partial(
    pl.kernel,
    out_type=[
        jax.ShapeDtypeStruct((B * N, C), jnp.bfloat16),
        jax.ShapeDtypeStruct((B, 2, C), jnp.float32),
    ],
    mesh=plsc.VectorSubcoreMesh(core_axis_name="c", subcore_axis_name="s"),
    compiler_params=pltpu.CompilerParams(use_tc_tiling_on_sc=False,
                                         needs_layout_passes=False),
    scratch_types=[
        pltpu.VMEM((N, CS), jnp.bfloat16),
        pltpu.VMEM((2, (NH + 1) * K), jnp.int32),
        pltpu.VMEM((2, NH, CS), jnp.bfloat16),
        pltpu.VMEM((2, CS), jnp.float32),
        pltpu.SemaphoreType.DMA,
        pltpu.SemaphoreType.DMA,
        pltpu.SemaphoreType.DMA,
        pltpu.SemaphoreType.DMA,
        pltpu.SemaphoreType.DMA,
    ],
)
def _sc_gather_max(x_hbm, idx_hbm, pooled_hbm, part_hbm,
                   xs_v, idx_v, pooled_v, stats_v,
                   sem_x, sem_i0, sem_i1, sem_o0, sem_o1):
    t = lax.axis_index("s") * 2 + lax.axis_index("c")
    b = t // NSL
    c0 = (t % NSL) * CS
    sem_i = (sem_i0, sem_i1)
    sem_o = (sem_o0, sem_o1)
    nch = N // NH

    def idx_fetch(h):
        return pltpu.async_copy(
            idx_hbm.at[pl.ds((b * N + h * NH) * K, NH * K)],
            idx_v.at[h % 2, pl.ds(0, NH * K)], sem_i[h % 2])

    cp_x = pltpu.async_copy(x_hbm.at[pl.ds(b * N, N), pl.ds(c0, CS)],
                            xs_v, sem_x)
    cp_i = idx_fetch(0)
    cp_x.wait()
    z = jnp.zeros((16,), jnp.float32)
    stats = (z,) * 8
    wb = [None, None]
    for h in range(nch):
        cur = h % 2
        cp_i.wait()
        if h + 1 < nch:
            cp_i = idx_fetch(h + 1)
        if wb[cur] is not None:
            wb[cur].wait()
        iv0 = idx_v[cur, pl.ds(0, K)]
        init = tuple(iv0[j] for j in range(K))

        def body(n, carry, h=h, cur=cur):
            a, st = carry
            ivn = idx_v[cur, pl.ds((n + 1) * K, K)]
            nxt = tuple(ivn[j] for j in range(K))
            m0 = xs_v[a[0], pl.ds(0, HC)]
            m1 = xs_v[a[0], pl.ds(HC, HC)]
            for j in range(1, K):
                m0 = jnp.maximum(m0, xs_v[a[j], pl.ds(0, HC)])
                m1 = jnp.maximum(m1, xs_v[a[j], pl.ds(HC, HC)])
            sn = h * NH + n
            p0 = m0 - xs_v[sn, pl.ds(0, HC)]
            p1 = m1 - xs_v[sn, pl.ds(HC, HC)]
            pooled_v[cur, n, pl.ds(0, HC)] = p0
            pooled_v[cur, n, pl.ds(HC, HC)] = p1
            e0, o0 = _split_f32(p0)
            e1, o1 = _split_f32(p1)
            se0, so0, se1, so1, qe0, qo0, qe1, qo1 = st
            st = (se0 + e0, so0 + o0, se1 + e1, so1 + o1,
                  qe0 + e0 * e0, qo0 + o0 * o0, qe1 + e1 * e1, qo1 + o1 * o1)
            return (nxt, st)

        _, stats = lax.fori_loop(0, NH, body, (init, stats))
        wb[cur] = pltpu.async_copy(
            pooled_v.at[cur],
            pooled_hbm.at[pl.ds(b * N + h * NH, NH), pl.ds(c0, CS)],
            sem_o[cur])
    for w in wb:
        w.wait()
    ii = jnp.arange(16, dtype=jnp.int32)
    ev, od = ii * 2, ii * 2 + 1
    r0, r1 = ii * 0, ii * 0 + 1
    se0, so0, se1, so1, qe0, qo0, qe1, qo1 = stats
    plsc.store_scatter(stats_v, [r0, ev], se0)
    plsc.store_scatter(stats_v, [r0, od], so0)
    plsc.store_scatter(stats_v, [r0, ev + HC], se1)
    plsc.store_scatter(stats_v, [r0, od + HC], so1)
    plsc.store_scatter(stats_v, [r1, ev], qe0)
    plsc.store_scatter(stats_v, [r1, od], qo0)
    plsc.store_scatter(stats_v, [r1, ev + HC], qe1)
    plsc.store_scatter(stats_v, [r1, od + HC], qo1)
    pltpu.sync_copy(stats_v, part_hbm.at[b, :, pl.ds(c0, CS)])


def _bn_body(pooled_ref, part_ref, w_ref, b_ref, o_ref):
    cnt = float(B * N)
    tot = jnp.sum(part_ref[...], axis=0)
    mean = tot[0:1, :] / cnt
    var = tot[1:2, :] / cnt - mean * mean
    inv = lax.rsqrt(var + 1e-5)
    o_ref[...] = (pooled_ref[...].astype(jnp.float32) - mean) \
        * (inv * w_ref[...]) + b_ref[...]


def _batchnorm(pooled2d, partials, bnw2d, bnb2d):
    blk = 4096
    nb = pooled2d.shape[0] // blk
    return pl.pallas_call(
        _bn_body,
        grid=(nb,),
        in_specs=[
            pl.BlockSpec((blk, C), lambda i: (i, 0)),
            pl.BlockSpec((B, 2, C), lambda i: (0, 0, 0)),
            pl.BlockSpec((1, C), lambda i: (0, 0)),
            pl.BlockSpec((1, C), lambda i: (0, 0)),
        ],
        out_specs=pl.BlockSpec((blk, C), lambda i: (i, 0)),
        out_shape=jax.ShapeDtypeStruct((pooled2d.shape[0], C), jnp.float32),
    )(pooled2d, partials, bnw2d, bnb2d)


def kernel(f, group_idx, W, bn_weight, bn_bias):
    x = _matmul(f.reshape(B * N, C), W)
    pooled, partials = _sc_gather_max(x, group_idx.reshape(B * N * K))
    out = _batchnorm(pooled, partials,
                     bn_weight.reshape(1, C), bn_bias.reshape(1, C))
    return out.reshape(B, N, C)
